# Initial kernel scaffold; baseline (speedup 1.0000x reference)
import functools

import jax
import jax.numpy as jnp
from jax import lax
from jax.experimental import pallas as pl
from jax.experimental.pallas import tpu as pltpu
from jax.experimental.pallas import tpu_sc as plsc


def _pack_table(embeddings, W):
    V, EMB = embeddings.shape
    VB = 2048
    Wpad = jnp.zeros((8, EMB), jnp.float32).at[: W.shape[0]].set(W)

    def body(w_ref, e_ref, o_ref):
        P = lax.dot_general(
            w_ref[...],
            e_ref[...],
            (((1,), (1,)), ((), ())),
            preferred_element_type=jnp.float32,
            precision=lax.Precision.HIGHEST,
        )
        u = lax.bitcast_convert_type(P, jnp.int32)
        r0 = (u[0:1] + 0x8000) & jnp.int32(-65536)
        r1 = ((u[1:2] + 0x8000) >> 16) & jnp.int32(0xFFFF)
        o_ref[...] = r0 | r1

    packed = pl.pallas_call(
        body,
        grid=(pl.cdiv(V, VB),),
        in_specs=[
            pl.BlockSpec((8, EMB), lambda i: (0, 0)),
            pl.BlockSpec((VB, EMB), lambda i: (i, 0)),
        ],
        out_specs=pl.BlockSpec((1, VB), lambda i: (0, i)),
        out_shape=jax.ShapeDtypeStruct((1, V), jnp.int32),
    )(Wpad, embeddings)
    return packed.reshape(V)


def _make_sc_pool(V, SEQ, B):
    NC, NS, L = 2, 16, 16
    NW = NC * NS
    BPW = B // NW
    S_CHUNK = 20
    NCH = SEQ // S_CHUNK
    NBG = BPW // L

    mesh = plsc.VectorSubcoreMesh(core_axis_name="c", subcore_axis_name="s")

    @functools.partial(
        pl.kernel,
        mesh=mesh,
        out_type=jax.ShapeDtypeStruct((2, B), jnp.float32),
        scratch_types=[
            pltpu.VMEM((V,), jnp.int32),
            pltpu.VMEM((2, S_CHUNK, BPW), jnp.int32),
            pltpu.VMEM((BPW,), jnp.float32),
            pltpu.VMEM((BPW,), jnp.float32),
            pltpu.VMEM((BPW,), jnp.int32),
            pltpu.VMEM((2, L), jnp.float32),
            pltpu.SemaphoreType.DMA,
            pltpu.SemaphoreType.DMA,
            pltpu.SemaphoreType.DMA,
            pltpu.SemaphoreType.DMA,
        ],
    )
    def pool(pp, xh, lh, bh, out, table_v, idx_v, acc0_v, acc1_v, len_v,
             bias_v, sem_t, sem_l, sem_a, sem_b):
        wid = lax.axis_index("s") * NC + lax.axis_index("c")
        base = wid * BPW
        sems = [sem_a, sem_b]

        cp_t = pltpu.async_copy(pp, table_v, sem_t)
        cp_l = pltpu.async_copy(lh.at[pl.ds(base, BPW)], len_v, sem_l)
        pltpu.sync_copy(bh, bias_v)
        cps = [None, None]
        cps[0] = pltpu.async_copy(
            xh.at[pl.ds(0, S_CHUNK), pl.ds(base, BPW)], idx_v.at[0], sem_a
        )
        cp_t.wait()

        for t in range(NCH):
            nxt = t + 1
            if nxt < NCH:
                cps[nxt % 2] = pltpu.async_copy(
                    xh.at[pl.ds(nxt * S_CHUNK, S_CHUNK), pl.ds(base, BPW)],
                    idx_v.at[nxt % 2],
                    sems[nxt % 2],
                )
            cps[t % 2].wait()
            buf = t % 2

            def bg_body(bg, carry, t=t, buf=buf):
                sl = pl.ds(bg * L, L)
                a0 = jnp.zeros((L,), jnp.float32)
                a1 = jnp.zeros((L,), jnp.float32)
                for si in range(S_CHUNK):
                    idx = idx_v[buf, si, sl]
                    g = plsc.load_gather(table_v, [idx])
                    a0 = a0 + plsc.bitcast(g & jnp.int32(-65536), jnp.float32)
                    a1 = a1 + plsc.bitcast(g << 16, jnp.float32)
                if t == 0:
                    acc0_v[sl] = a0
                    acc1_v[sl] = a1
                else:
                    acc0_v[sl] = acc0_v[sl] + a0
                    acc1_v[sl] = acc1_v[sl] + a1
                return carry

            lax.fori_loop(0, NBG, bg_body, 0)

        cp_l.wait()
        b0 = bias_v[0]
        b1 = bias_v[1]

        def fin(bg, carry):
            sl = pl.ds(bg * L, L)
            linv = 1.0 / len_v[sl].astype(jnp.float32)
            acc0_v[sl] = acc0_v[sl] * linv + b0
            acc1_v[sl] = acc1_v[sl] * linv + b1
            return carry

        lax.fori_loop(0, NBG, fin, 0)
        pltpu.sync_copy(acc0_v, out.at[0, pl.ds(base, BPW)])
        pltpu.sync_copy(acc1_v, out.at[1, pl.ds(base, BPW)])

    return pool


def kernel(x, x_lengths, embeddings, W, b):
    SEQ, B = x.shape
    V, _EMB = embeddings.shape
    packed = _pack_table(embeddings, W)
    bias2 = jnp.broadcast_to(b.astype(jnp.float32)[:, None], (2, 16))
    pool = _make_sc_pool(V, SEQ, B)
    out2 = pool(packed, x, x_lengths, bias2)
    return out2.T


# speedup vs baseline: 86.8273x; 86.8273x over previous
import functools

import jax
import jax.numpy as jnp
from jax import lax
from jax.experimental import pallas as pl
from jax.experimental.pallas import tpu as pltpu
from jax.experimental.pallas import tpu_sc as plsc


def _pack_table(embeddings, W):
    V, EMB = embeddings.shape
    VB = 2048
    Wpad = jnp.zeros((8, EMB), jnp.float32).at[: W.shape[0]].set(W)

    def body(w_ref, e_ref, o_ref):
        P = lax.dot_general(
            w_ref[...],
            e_ref[...],
            (((1,), (1,)), ((), ())),
            preferred_element_type=jnp.float32,
            precision=lax.Precision.HIGHEST,
        )
        u = lax.bitcast_convert_type(P, jnp.int32)
        r0 = (u[0:1] + 0x8000) & jnp.int32(-65536)
        r1 = ((u[1:2] + 0x8000) >> 16) & jnp.int32(0xFFFF)
        o_ref[...] = r0 | r1

    packed = pl.pallas_call(
        body,
        grid=(pl.cdiv(V, VB),),
        in_specs=[
            pl.BlockSpec((8, EMB), lambda i: (0, 0)),
            pl.BlockSpec((VB, EMB), lambda i: (i, 0)),
        ],
        out_specs=pl.BlockSpec((1, VB), lambda i: (0, i)),
        out_shape=jax.ShapeDtypeStruct((1, V), jnp.int32),
    )(Wpad, embeddings)
    return packed.reshape(V)


def _make_sc_pool(V, SEQ, B):
    NC, NS, L = 2, 16, 16
    NW = NC * NS
    BPW = B // NW
    S_CHUNK = 8
    NCH = SEQ // S_CHUNK
    NBG = BPW // L

    mesh = plsc.VectorSubcoreMesh(core_axis_name="c", subcore_axis_name="s")

    @functools.partial(
        pl.kernel,
        mesh=mesh,
        compiler_params=pltpu.CompilerParams(needs_layout_passes=False),
        out_type=jax.ShapeDtypeStruct((2, B), jnp.float32),
        scratch_types=[
            pltpu.VMEM((V,), jnp.int32),
            pltpu.VMEM((2, S_CHUNK, BPW), jnp.int32),
            pltpu.VMEM((BPW,), jnp.float32),
            pltpu.VMEM((BPW,), jnp.float32),
            pltpu.VMEM((BPW,), jnp.int32),
            pltpu.VMEM((2, L), jnp.float32),
            pltpu.SemaphoreType.DMA,
            pltpu.SemaphoreType.DMA,
            pltpu.SemaphoreType.DMA,
            pltpu.SemaphoreType.DMA,
        ],
    )
    def pool(pp, xh, lh, bh, out, table_v, idx_v, acc0_v, acc1_v, len_v,
             bias_v, sem_t, sem_l, sem_a, sem_b):
        wid = lax.axis_index("s") * NC + lax.axis_index("c")
        base = wid * BPW
        sems = [sem_a, sem_b]

        cp_t = pltpu.async_copy(pp, table_v, sem_t)
        cp_l = pltpu.async_copy(lh.at[pl.ds(base, BPW)], len_v, sem_l)
        pltpu.sync_copy(bh, bias_v)
        cps = [None, None]
        cps[0] = pltpu.async_copy(
            xh.at[pl.ds(0, S_CHUNK), pl.ds(base, BPW)], idx_v.at[0], sem_a
        )
        cp_t.wait()

        for t in range(NCH):
            nxt = t + 1
            if nxt < NCH:
                cps[nxt % 2] = pltpu.async_copy(
                    xh.at[pl.ds(nxt * S_CHUNK, S_CHUNK), pl.ds(base, BPW)],
                    idx_v.at[nxt % 2],
                    sems[nxt % 2],
                )
            cps[t % 2].wait()
            buf = t % 2

            def bg_body(bg, carry, t=t, buf=buf):
                sl = pl.ds(bg * L, L)
                a0 = jnp.zeros((L,), jnp.float32)
                a1 = jnp.zeros((L,), jnp.float32)
                for si in range(S_CHUNK):
                    idx = idx_v[buf, si, sl]
                    g = plsc.load_gather(table_v, [idx])
                    a0 = a0 + plsc.bitcast(g & jnp.int32(-65536), jnp.float32)
                    a1 = a1 + plsc.bitcast(g << 16, jnp.float32)
                if t == 0:
                    acc0_v[sl] = a0
                    acc1_v[sl] = a1
                else:
                    acc0_v[sl] = acc0_v[sl] + a0
                    acc1_v[sl] = acc1_v[sl] + a1
                return carry

            lax.fori_loop(0, NBG, bg_body, 0)

        cp_l.wait()
        b0 = bias_v[0]
        b1 = bias_v[1]

        def fin(bg, carry):
            sl = pl.ds(bg * L, L)
            linv = 1.0 / len_v[sl].astype(jnp.float32)
            acc0_v[sl] = acc0_v[sl] * linv + b0
            acc1_v[sl] = acc1_v[sl] * linv + b1
            return carry

        lax.fori_loop(0, NBG, fin, 0)
        pltpu.sync_copy(acc0_v, out.at[0, pl.ds(base, BPW)])
        pltpu.sync_copy(acc1_v, out.at[1, pl.ds(base, BPW)])

    return pool


def kernel(x, x_lengths, embeddings, W, b):
    SEQ, B = x.shape
    V, _EMB = embeddings.shape
    packed = _pack_table(embeddings, W)
    bias2 = jnp.broadcast_to(b.astype(jnp.float32)[:, None], (2, 16))
    pool = _make_sc_pool(V, SEQ, B)
    out2 = pool(packed, x, x_lengths, bias2)
    return out2.T


# speedup vs baseline: 104.9846x; 1.2091x over previous
import functools

import jax
import jax.numpy as jnp
from jax import lax
from jax.experimental import pallas as pl
from jax.experimental.pallas import tpu as pltpu
from jax.experimental.pallas import tpu_sc as plsc


def _pack_table(embeddings, W):
    V, EMB = embeddings.shape
    VB = 8192
    Wpad = jnp.zeros((8, EMB), jnp.float32).at[: W.shape[0]].set(W)

    def body(w_ref, e_ref, o_ref):
        P = lax.dot_general(
            w_ref[...].astype(jnp.bfloat16),
            e_ref[...].astype(jnp.bfloat16),
            (((1,), (1,)), ((), ())),
            preferred_element_type=jnp.float32,
        )
        u = lax.bitcast_convert_type(P, jnp.int32)
        r0 = (u[0:1] + 0x8000) & jnp.int32(-65536)
        r1 = ((u[1:2] + 0x8000) >> 16) & jnp.int32(0xFFFF)
        o_ref[...] = r0 | r1

    packed = pl.pallas_call(
        body,
        grid=(pl.cdiv(V, VB),),
        in_specs=[
            pl.BlockSpec((8, EMB), lambda i: (0, 0)),
            pl.BlockSpec((VB, EMB), lambda i: (i, 0)),
        ],
        out_specs=pl.BlockSpec((1, VB), lambda i: (0, i)),
        out_shape=jax.ShapeDtypeStruct((1, V), jnp.int32),
    )(Wpad, embeddings)
    return packed.reshape(V)


def _make_sc_pool(V, SEQ, B):
    NC, NS, L = 2, 16, 16
    NW = NC * NS
    BPW = B // NW
    S_CHUNK = 16
    chunk_sizes = []
    s_off = 0
    while s_off < SEQ:
        c = min(S_CHUNK, SEQ - s_off)
        chunk_sizes.append(c)
        s_off += c
    NBG = BPW // L

    mesh = plsc.VectorSubcoreMesh(core_axis_name="c", subcore_axis_name="s")

    @functools.partial(
        pl.kernel,
        mesh=mesh,
        compiler_params=pltpu.CompilerParams(needs_layout_passes=False),
        out_type=jax.ShapeDtypeStruct((2 * B,), jnp.float32),
        scratch_types=[
            pltpu.VMEM((V,), jnp.int32),
            pltpu.VMEM((2, S_CHUNK, BPW), jnp.int32),
            pltpu.VMEM((BPW,), jnp.float32),
            pltpu.VMEM((BPW,), jnp.float32),
            pltpu.VMEM((BPW,), jnp.int32),
            pltpu.VMEM((2, L), jnp.float32),
            pltpu.VMEM((2 * BPW,), jnp.float32),
            pltpu.SemaphoreType.DMA,
            pltpu.SemaphoreType.DMA,
            pltpu.SemaphoreType.DMA,
            pltpu.SemaphoreType.DMA,
        ],
    )
    def pool(pp, xh, lh, bh, out, table_v, idx_v, acc0_v, acc1_v, len_v,
             bias_v, outp_v, sem_t, sem_l, sem_a, sem_b):
        wid = lax.axis_index("s") * NC + lax.axis_index("c")
        base = wid * BPW
        sems = [sem_a, sem_b]

        cp_t = pltpu.async_copy(pp, table_v, sem_t)
        cp_l = pltpu.async_copy(lh.at[pl.ds(base, BPW)], len_v, sem_l)
        pltpu.sync_copy(bh, bias_v)
        cps = [None, None]
        cps[0] = pltpu.async_copy(
            xh.at[pl.ds(0, chunk_sizes[0]), pl.ds(base, BPW)],
            idx_v.at[0, pl.ds(0, chunk_sizes[0])],
            sem_a,
        )
        cp_t.wait()

        s_base = 0
        for t, csz in enumerate(chunk_sizes):
            nxt = t + 1
            if nxt < len(chunk_sizes):
                cps[nxt % 2] = pltpu.async_copy(
                    xh.at[pl.ds(s_base + csz, chunk_sizes[nxt]), pl.ds(base, BPW)],
                    idx_v.at[nxt % 2, pl.ds(0, chunk_sizes[nxt])],
                    sems[nxt % 2],
                )
            cps[t % 2].wait()
            buf = t % 2

            def bg_body(bg, carry, t=t, buf=buf, csz=csz):
                sl = pl.ds(bg * L, L)
                a0 = jnp.zeros((L,), jnp.float32)
                a1 = jnp.zeros((L,), jnp.float32)
                for si in range(csz):
                    idx = idx_v[buf, si, sl]
                    g = plsc.load_gather(table_v, [idx])
                    a0 = a0 + plsc.bitcast(g & jnp.int32(-65536), jnp.float32)
                    a1 = a1 + plsc.bitcast(g << 16, jnp.float32)
                if t == 0:
                    acc0_v[sl] = a0
                    acc1_v[sl] = a1
                else:
                    acc0_v[sl] = acc0_v[sl] + a0
                    acc1_v[sl] = acc1_v[sl] + a1
                return carry

            lax.fori_loop(0, NBG, bg_body, 0)
            s_base += csz

        cp_l.wait()
        b0 = bias_v[0]
        b1 = bias_v[1]
        iota2 = lax.iota(jnp.int32, 16) * 2

        def fin(bg, carry):
            sl = pl.ds(bg * L, L)
            linv = 1.0 / len_v[sl].astype(jnp.float32)
            pos = iota2 + bg * (2 * L)
            plsc.store_scatter(outp_v, [pos], acc0_v[sl] * linv + b0)
            plsc.store_scatter(outp_v, [pos + 1], acc1_v[sl] * linv + b1)
            return carry

        lax.fori_loop(0, NBG, fin, 0)
        pltpu.sync_copy(outp_v, out.at[pl.ds(2 * base, 2 * BPW)])

    return pool


def kernel(x, x_lengths, embeddings, W, b):
    SEQ, B = x.shape
    V, _EMB = embeddings.shape
    packed = _pack_table(embeddings, W)
    bias2 = jnp.broadcast_to(b.astype(jnp.float32)[:, None], (2, 16))
    pool = _make_sc_pool(V, SEQ, B)
    flat = pool(packed, x, x_lengths, bias2)
    return flat.reshape(B, 2)


# speedup vs baseline: 207.6904x; 1.9783x over previous
import functools

import jax
import jax.numpy as jnp
from jax import lax
from jax.experimental import pallas as pl
from jax.experimental.pallas import tpu as pltpu
from jax.experimental.pallas import tpu_sc as plsc


def _pack_table(embeddings, W):
    V, EMB = embeddings.shape
    VB = 8192
    Wpad = jnp.zeros((8, EMB), jnp.float32).at[: W.shape[0]].set(W)
    et = embeddings.T

    def body(w_ref, e_ref, o_ref):
        P = lax.dot_general(
            w_ref[...].astype(jnp.bfloat16),
            e_ref[...].astype(jnp.bfloat16),
            (((1,), (0,)), ((), ())),
            preferred_element_type=jnp.float32,
        )
        u = lax.bitcast_convert_type(P, jnp.int32)
        r0 = (u[0:1] + 0x8000) & jnp.int32(-65536)
        r1 = ((u[1:2] + 0x8000) >> 16) & jnp.int32(0xFFFF)
        o_ref[...] = jnp.reshape(r0 | r1, (VB,))

    return pl.pallas_call(
        body,
        grid=(pl.cdiv(V, VB),),
        in_specs=[
            pl.BlockSpec((8, EMB), lambda i: (0, 0)),
            pl.BlockSpec((EMB, VB), lambda i: (0, i)),
        ],
        out_specs=pl.BlockSpec((VB,), lambda i: (i,)),
        out_shape=jax.ShapeDtypeStruct((V,), jnp.int32),
    )(Wpad, et)


def _make_sc_pool(V, SEQ, B):
    NC, NS, L = 2, 16, 16
    NW = NC * NS
    BPW = B // NW
    S_CHUNK = 16
    chunk_sizes = []
    s_off = 0
    while s_off < SEQ:
        c = min(S_CHUNK, SEQ - s_off)
        chunk_sizes.append(c)
        s_off += c
    NBG = BPW // L

    mesh = plsc.VectorSubcoreMesh(core_axis_name="c", subcore_axis_name="s")

    @functools.partial(
        pl.kernel,
        mesh=mesh,
        compiler_params=pltpu.CompilerParams(needs_layout_passes=False),
        out_type=jax.ShapeDtypeStruct((2, B), jnp.float32),
        scratch_types=[
            pltpu.VMEM((V,), jnp.int32),
            pltpu.VMEM((2, S_CHUNK, BPW), jnp.int32),
            pltpu.VMEM((BPW,), jnp.float32),
            pltpu.VMEM((BPW,), jnp.float32),
            pltpu.VMEM((BPW,), jnp.int32),
            pltpu.VMEM((2, L), jnp.float32),
            pltpu.SemaphoreType.DMA,
            pltpu.SemaphoreType.DMA,
            pltpu.SemaphoreType.DMA,
            pltpu.SemaphoreType.DMA,
        ],
    )
    def pool(pp, xh, lh, bh, out, table_v, idx_v, acc0_v, acc1_v, len_v,
             bias_v, sem_t, sem_l, sem_a, sem_b):
        wid = lax.axis_index("s") * NC + lax.axis_index("c")
        base = wid * BPW
        sems = [sem_a, sem_b]

        cp_t = pltpu.async_copy(pp, table_v, sem_t)
        cp_l = pltpu.async_copy(lh.at[pl.ds(base, BPW)], len_v, sem_l)
        pltpu.sync_copy(bh, bias_v)
        cps = [None, None]
        cps[0] = pltpu.async_copy(
            xh.at[pl.ds(0, chunk_sizes[0]), pl.ds(base, BPW)],
            idx_v.at[0, pl.ds(0, chunk_sizes[0])],
            sem_a,
        )
        cp_t.wait()

        s_base = 0
        for t, csz in enumerate(chunk_sizes):
            nxt = t + 1
            if nxt < len(chunk_sizes):
                cps[nxt % 2] = pltpu.async_copy(
                    xh.at[pl.ds(s_base + csz, chunk_sizes[nxt]), pl.ds(base, BPW)],
                    idx_v.at[nxt % 2, pl.ds(0, chunk_sizes[nxt])],
                    sems[nxt % 2],
                )
            cps[t % 2].wait()
            buf = t % 2

            def bg_body(bg, carry, t=t, buf=buf, csz=csz):
                sl = pl.ds(bg * L, L)
                a0 = jnp.zeros((L,), jnp.float32)
                a1 = jnp.zeros((L,), jnp.float32)
                for si in range(csz):
                    idx = idx_v[buf, si, sl]
                    g = plsc.load_gather(table_v, [idx])
                    a0 = a0 + plsc.bitcast(g & jnp.int32(-65536), jnp.float32)
                    a1 = a1 + plsc.bitcast(g << 16, jnp.float32)
                if t == 0:
                    acc0_v[sl] = a0
                    acc1_v[sl] = a1
                else:
                    acc0_v[sl] = acc0_v[sl] + a0
                    acc1_v[sl] = acc1_v[sl] + a1
                return carry

            lax.fori_loop(0, NBG, bg_body, 0)
            s_base += csz

        cp_l.wait()
        b0 = bias_v[0]
        b1 = bias_v[1]
        def fin(bg, carry):
            sl = pl.ds(bg * L, L)
            linv = 1.0 / len_v[sl].astype(jnp.float32)
            acc0_v[sl] = acc0_v[sl] * linv + b0
            acc1_v[sl] = acc1_v[sl] * linv + b1
            return carry

        lax.fori_loop(0, NBG, fin, 0)
        pltpu.sync_copy(acc0_v, out.at[0, pl.ds(base, BPW)])
        pltpu.sync_copy(acc1_v, out.at[1, pl.ds(base, BPW)])

    return pool


def kernel(x, x_lengths, embeddings, W, b):
    SEQ, B = x.shape
    V, _EMB = embeddings.shape
    packed = _pack_table(embeddings, W)
    bias2 = jnp.broadcast_to(b.astype(jnp.float32)[:, None], (2, 16))
    pool = _make_sc_pool(V, SEQ, B)
    out2 = pool(packed, x, x_lengths, bias2)
    return out2.T


# speedup vs baseline: 219.2386x; 1.0556x over previous
import functools

import jax
import jax.numpy as jnp
from jax import lax
from jax.experimental import pallas as pl
from jax.experimental.pallas import tpu as pltpu
from jax.experimental.pallas import tpu_sc as plsc


def _pack_table(embeddings, W):
    V, EMB = embeddings.shape
    VB = 8192
    Wpad = jnp.zeros((8, EMB), jnp.float32).at[: W.shape[0]].set(W)
    et = embeddings.T

    def body(w_ref, e_ref, o_ref):
        P = lax.dot_general(
            w_ref[...].astype(jnp.bfloat16),
            e_ref[...].astype(jnp.bfloat16),
            (((1,), (0,)), ((), ())),
            preferred_element_type=jnp.float32,
        )
        u = lax.bitcast_convert_type(P, jnp.int32)
        r0 = (u[0:1] + 0x8000) & jnp.int32(-65536)
        r1 = ((u[1:2] + 0x8000) >> 16) & jnp.int32(0xFFFF)
        o_ref[...] = jnp.reshape(r0 | r1, (VB,))

    return pl.pallas_call(
        body,
        grid=(pl.cdiv(V, VB),),
        in_specs=[
            pl.BlockSpec((8, EMB), lambda i: (0, 0)),
            pl.BlockSpec((EMB, VB), lambda i: (0, i)),
        ],
        out_specs=pl.BlockSpec((VB,), lambda i: (i,)),
        out_shape=jax.ShapeDtypeStruct((V,), jnp.int32),
    )(Wpad, et)


def _make_sc_pool(V, SEQ, B):
    NC, NS, L = 2, 16, 16
    NW = NC * NS
    BPW = B // NW
    S_CHUNK = 24
    chunk_sizes = []
    s_off = 0
    while s_off < SEQ:
        c = min(S_CHUNK, SEQ - s_off)
        chunk_sizes.append(c)
        s_off += c
    NBG = BPW // L

    mesh = plsc.VectorSubcoreMesh(core_axis_name="c", subcore_axis_name="s")

    @functools.partial(
        pl.kernel,
        mesh=mesh,
        compiler_params=pltpu.CompilerParams(needs_layout_passes=False),
        out_type=jax.ShapeDtypeStruct((2, B), jnp.float32),
        scratch_types=[
            pltpu.VMEM((V,), jnp.int32),
            pltpu.VMEM((2, S_CHUNK, BPW), jnp.int32),
            pltpu.VMEM((BPW,), jnp.float32),
            pltpu.VMEM((BPW,), jnp.float32),
            pltpu.VMEM((BPW,), jnp.int32),
            pltpu.VMEM((2, L), jnp.float32),
            pltpu.SemaphoreType.DMA,
            pltpu.SemaphoreType.DMA,
            pltpu.SemaphoreType.DMA,
            pltpu.SemaphoreType.DMA,
        ],
    )
    def pool(pp, xh, lh, bh, out, table_v, idx_v, acc0_v, acc1_v, len_v,
             bias_v, sem_t, sem_l, sem_a, sem_b):
        wid = lax.axis_index("s") * NC + lax.axis_index("c")
        base = wid * BPW
        sems = [sem_a, sem_b]

        cp_t = pltpu.async_copy(pp, table_v, sem_t)
        cp_l = pltpu.async_copy(lh.at[pl.ds(base, BPW)], len_v, sem_l)
        pltpu.sync_copy(bh, bias_v)
        cps = [None, None]
        cps[0] = pltpu.async_copy(
            xh.at[pl.ds(0, chunk_sizes[0]), pl.ds(base, BPW)],
            idx_v.at[0, pl.ds(0, chunk_sizes[0])],
            sem_a,
        )
        cp_t.wait()

        s_base = 0
        for t, csz in enumerate(chunk_sizes):
            nxt = t + 1
            if nxt < len(chunk_sizes):
                cps[nxt % 2] = pltpu.async_copy(
                    xh.at[pl.ds(s_base + csz, chunk_sizes[nxt]), pl.ds(base, BPW)],
                    idx_v.at[nxt % 2, pl.ds(0, chunk_sizes[nxt])],
                    sems[nxt % 2],
                )
            cps[t % 2].wait()
            buf = t % 2

            def bg_body(bg, carry, t=t, buf=buf, csz=csz):
                sl = pl.ds(bg * L, L)
                NP = 4
                z = jnp.zeros((L,), jnp.float32)
                p0 = [z] * NP
                p1 = [z] * NP
                for si in range(csz):
                    idx = idx_v[buf, si, sl]
                    g = plsc.load_gather(table_v, [idx])
                    j = si % NP
                    p0[j] = p0[j] + plsc.bitcast(g & jnp.int32(-65536), jnp.float32)
                    p1[j] = p1[j] + plsc.bitcast(g << 16, jnp.float32)
                a0 = (p0[0] + p0[1]) + (p0[2] + p0[3])
                a1 = (p1[0] + p1[1]) + (p1[2] + p1[3])
                if t == 0:
                    acc0_v[sl] = a0
                    acc1_v[sl] = a1
                else:
                    acc0_v[sl] = acc0_v[sl] + a0
                    acc1_v[sl] = acc1_v[sl] + a1
                return carry

            lax.fori_loop(0, NBG, bg_body, 0)
            s_base += csz

        cp_l.wait()
        b0 = bias_v[0]
        b1 = bias_v[1]
        def fin(bg, carry):
            sl = pl.ds(bg * L, L)
            linv = 1.0 / len_v[sl].astype(jnp.float32)
            acc0_v[sl] = acc0_v[sl] * linv + b0
            acc1_v[sl] = acc1_v[sl] * linv + b1
            return carry

        lax.fori_loop(0, NBG, fin, 0)
        pltpu.sync_copy(acc0_v, out.at[0, pl.ds(base, BPW)])
        pltpu.sync_copy(acc1_v, out.at[1, pl.ds(base, BPW)])

    return pool


def kernel(x, x_lengths, embeddings, W, b):
    SEQ, B = x.shape
    V, _EMB = embeddings.shape
    packed = _pack_table(embeddings, W)
    bias2 = jnp.broadcast_to(b.astype(jnp.float32)[:, None], (2, 16))
    pool = _make_sc_pool(V, SEQ, B)
    out2 = pool(packed, x, x_lengths, bias2)
    return out2.T


# speedup vs baseline: 254.0753x; 1.1589x over previous
import functools

import jax
import jax.numpy as jnp
from jax import lax
from jax.experimental import pallas as pl
from jax.experimental.pallas import tpu as pltpu
from jax.experimental.pallas import tpu_sc as plsc


def _pack_table(embeddings, W):
    V, EMB = embeddings.shape
    VB = 8192
    et = embeddings.T

    def body(w_ref, e_ref, o_ref):
        P = lax.dot_general(
            w_ref[...].astype(jnp.bfloat16),
            e_ref[...].astype(jnp.bfloat16),
            (((1,), (0,)), ((), ())),
            preferred_element_type=jnp.float32,
        )
        u = lax.bitcast_convert_type(P, jnp.int32)
        r0 = (u[0:1] + 0x8000) & jnp.int32(-65536)
        r1 = ((u[1:2] + 0x8000) >> 16) & jnp.int32(0xFFFF)
        o_ref[...] = jnp.reshape(r0 | r1, (VB,))

    return pl.pallas_call(
        body,
        grid=(pl.cdiv(V, VB),),
        in_specs=[
            pl.BlockSpec((2, EMB), lambda i: (0, 0)),
            pl.BlockSpec((EMB, VB), lambda i: (0, i)),
        ],
        out_specs=pl.BlockSpec((VB,), lambda i: (i,)),
        out_shape=jax.ShapeDtypeStruct((V,), jnp.int32),
    )(W, et)


def _make_sc_pool(V, SEQ, B):
    NC, NS, L = 2, 16, 16
    NW = NC * NS
    BPW = B // NW
    S_CHUNK = 16
    chunk_sizes = []
    s_off = 0
    while s_off < SEQ:
        c = min(S_CHUNK, SEQ - s_off)
        chunk_sizes.append(c)
        s_off += c
    NBG = BPW // L

    mesh = plsc.VectorSubcoreMesh(core_axis_name="c", subcore_axis_name="s")

    @functools.partial(
        pl.kernel,
        mesh=mesh,
        compiler_params=pltpu.CompilerParams(needs_layout_passes=False),
        out_type=jax.ShapeDtypeStruct((2, B), jnp.float32),
        scratch_types=[
            pltpu.VMEM_SHARED((V,), jnp.int32),
            pltpu.VMEM((V,), jnp.int32),
            pltpu.VMEM((2, S_CHUNK, BPW), jnp.int32),
            pltpu.VMEM((BPW,), jnp.float32),
            pltpu.VMEM((BPW,), jnp.float32),
            pltpu.VMEM((BPW,), jnp.int32),
            pltpu.VMEM((2, L), jnp.float32),
            pltpu.SemaphoreType.DMA,
            pltpu.SemaphoreType.DMA,
            pltpu.SemaphoreType.DMA,
        ],
    )
    def pool(pp, xh, lh, bh, out, table_s, table_v, idx_v, acc0_v, acc1_v,
             len_v, bias_v, sem_l, sem_a, sem_b):
        cid = lax.axis_index("c")
        sid = lax.axis_index("s")
        wid = sid * NC + cid
        base = wid * BPW
        sems = [sem_a, sem_b]

        cp_l = pltpu.async_copy(lh.at[pl.ds(base, BPW)], len_v, sem_l)
        cps = [None, None]
        for j in range(min(2, len(chunk_sizes))):
            cps[j] = pltpu.async_copy(
                xh.at[pl.ds(j * S_CHUNK, chunk_sizes[j]), pl.ds(base, BPW)],
                idx_v.at[j, pl.ds(0, chunk_sizes[j])],
                sems[j],
            )
        with jax.named_scope("table_stage"):
            @pl.when(sid == 0)
            def _():
                pltpu.sync_copy(pp, table_s)

            plsc.subcore_barrier()
        with jax.named_scope("table_wait"):
            pltpu.sync_copy(table_s, table_v)

        s_base = 0
        for t, csz in enumerate(chunk_sizes):
            with jax.named_scope(f"idx_wait_{t}"):
                cps[t % 2].wait()
            buf = t % 2

            def bg_body(bg, t=t, buf=buf, csz=csz):
                sl = pl.ds(bg * L, L)
                NP = 4
                z = jnp.zeros((L,), jnp.float32)
                p0 = [z] * NP
                p1 = [z] * NP
                for si in range(csz):
                    idx = idx_v[buf, si, sl]
                    g = plsc.load_gather(table_v, [idx])
                    j = si % NP
                    p0[j] = p0[j] + plsc.bitcast(g & jnp.int32(-65536), jnp.float32)
                    p1[j] = p1[j] + plsc.bitcast(g << 16, jnp.float32)
                a0 = (p0[0] + p0[1]) + (p0[2] + p0[3])
                a1 = (p1[0] + p1[1]) + (p1[2] + p1[3])
                if t == 0:
                    acc0_v[sl] = a0
                    acc1_v[sl] = a1
                else:
                    acc0_v[sl] = acc0_v[sl] + a0
                    acc1_v[sl] = acc1_v[sl] + a1

            with jax.named_scope(f"gather_{t}"):
                plsc.parallel_loop(0, NBG, 1, unroll=2)(bg_body)
            s_base += csz
            nxt = t + 2
            if nxt < len(chunk_sizes):
                cps[t % 2] = pltpu.async_copy(
                    xh.at[pl.ds(nxt * S_CHUNK, chunk_sizes[nxt]), pl.ds(base, BPW)],
                    idx_v.at[t % 2, pl.ds(0, chunk_sizes[nxt])],
                    sems[t % 2],
                )

        pltpu.sync_copy(bh, bias_v)
        cp_l.wait()
        b0 = bias_v[0]
        b1 = bias_v[1]

        def fin(bg):
            sl = pl.ds(bg * L, L)
            linv = 1.0 / len_v[sl].astype(jnp.float32)
            acc0_v[sl] = acc0_v[sl] * linv + b0
            acc1_v[sl] = acc1_v[sl] * linv + b1

        with jax.named_scope("finalize"):
            plsc.parallel_loop(0, NBG, 1, unroll=2)(fin)
            pltpu.sync_copy(acc0_v, out.at[0, pl.ds(base, BPW)])
            pltpu.sync_copy(acc1_v, out.at[1, pl.ds(base, BPW)])

    return pool


def kernel(x, x_lengths, embeddings, W, b):
    SEQ, B = x.shape
    V, _EMB = embeddings.shape
    packed = _pack_table(embeddings, W)
    bias2 = jnp.broadcast_to(b.astype(jnp.float32)[:, None], (2, 16))
    pool = _make_sc_pool(V, SEQ, B)
    out2 = pool(packed, x, x_lengths, bias2)
    return out2.T
